# baseline (speedup 1.0000x reference)
import jax
import jax.numpy as jnp
from jax.experimental import pallas as pl
from jax.experimental.pallas import tpu as pltpu

_B = 16
_T = 16384
_H = 1024
_CHUNK = 2048
_K = _T // _CHUNK


def _body(starts_ref, ends_ref, yt_ref, yv_ref, mes_ref,
          s_ref, yv_out_ref, m_ref, z_ref):
    k = pl.program_id(0)

    @pl.when(k == 0)
    def _init():
        m_ref[...] = jnp.full((_B, 1), -1e9, dtype=jnp.float32)
        z_ref[...] = jnp.zeros((_B, 1), dtype=jnp.float32)
        s_ref[...] = jnp.zeros_like(s_ref)
        yv_out_ref[...] = jnp.zeros_like(yv_out_ref)

    starts = starts_ref[...]
    ends = ends_ref[...]
    sel = (jax.lax.broadcasted_iota(jnp.int32, (1, 4), 1) >= 1).astype(
        jnp.float32)
    score = jax.lax.dot_general(
        sel, yt_ref[...],
        dimension_numbers=(((1,), (1,)), ((), ())),
        preferred_element_type=jnp.float32,
    )
    pos = k * _CHUNK + jax.lax.broadcasted_iota(jnp.int32, (_B, _CHUNK), 1)
    mask = (pos >= starts) & (pos < ends)
    masked = jnp.where(mask, score, jnp.float32(-1e9))
    m_prev = m_ref[...]
    m_new = jnp.maximum(m_prev, jnp.max(masked, axis=1, keepdims=True))
    alpha = jnp.exp(m_prev - m_new)
    e = jnp.where(mask, jnp.exp(score - m_new), 0.0)
    m_ref[...] = m_new
    z_ref[...] = z_ref[...] * alpha + jnp.sum(e, axis=1, keepdims=True)
    s_ref[...] = s_ref[...] * alpha + jax.lax.dot_general(
        e, mes_ref[...],
        dimension_numbers=(((1,), (0,)), ((), ())),
        preferred_element_type=jnp.float32,
    )
    onehot = (pos == starts).astype(jnp.float32)
    yv_out_ref[...] += jax.lax.dot_general(
        onehot, yv_ref[...],
        dimension_numbers=(((1,), (0,)), ((), ())),
        preferred_element_type=jnp.float32,
    )

    @pl.when(k == _K - 1)
    def _fin():
        z = z_ref[...]
        s_ref[...] = s_ref[...] * jnp.where(z > 0.0, 1.0 / z, 0.0)


def kernel(mes_update, yv, yt_pred, cu_seqlens):
    starts = cu_seqlens[:-1].reshape(_B, 1)
    ends = cu_seqlens[1:].reshape(_B, 1)
    s, yv_cas = pl.pallas_call(
        _body,
        grid=(_K,),
        in_specs=[
            pl.BlockSpec((_B, 1), lambda k: (0, 0)),
            pl.BlockSpec((_B, 1), lambda k: (0, 0)),
            pl.BlockSpec((_CHUNK, 4), lambda k: (k, 0)),
            pl.BlockSpec((_CHUNK, 4), lambda k: (k, 0)),
            pl.BlockSpec((_CHUNK, _H), lambda k: (k, 0)),
        ],
        out_specs=(
            pl.BlockSpec((_B, _H), lambda k: (0, 0)),
            pl.BlockSpec((_B, 4), lambda k: (0, 0)),
        ),
        out_shape=(
            jax.ShapeDtypeStruct((_B, _H), jnp.float32),
            jax.ShapeDtypeStruct((_B, 4), jnp.float32),
        ),
        scratch_shapes=[
            pltpu.VMEM((_B, 1), jnp.float32),
            pltpu.VMEM((_B, 1), jnp.float32),
        ],
        compiler_params=pltpu.CompilerParams(
            dimension_semantics=("arbitrary",),
        ),
    )(starts, ends, yt_pred, yv, mes_update)
    return (s, yv_cas)


# speedup vs baseline: 1.4660x; 1.4660x over previous
import jax
import jax.numpy as jnp
from jax.experimental import pallas as pl
from jax.experimental.pallas import tpu as pltpu

_B = 16
_T = 16384
_H = 1024
_CHUNK = 2048
_K = _T // _CHUNK


def _body(starts_ref, ends_ref, ytv_ref, mes_ref, s_ref, yv_out_ref, w_ref):
    k = pl.program_id(0)

    @pl.when(k == 0)
    def _init():
        starts = starts_ref[...]
        ends = ends_ref[...]
        ytv = ytv_ref[...]
        score = ytv[5:6, :] + ytv[6:7, :] + ytv[7:8, :]
        pos = jax.lax.broadcasted_iota(jnp.int32, (_B, _T), 1)
        mask = (pos >= starts) & (pos < ends)
        masked = jnp.where(mask, score, jnp.float32(-1e9))
        m = jnp.max(masked, axis=1, keepdims=True)
        e = jnp.where(mask, jnp.exp(score - m), 0.0)
        z = jnp.sum(e, axis=1, keepdims=True)
        w_ref[...] = e * jnp.where(z > 0.0, 1.0 / z, 0.0)
        onehot = (pos == starts).astype(jnp.float32)
        yv_out_ref[...] = jax.lax.dot_general(
            onehot, ytv,
            dimension_numbers=(((1,), (1,)), ((), ())),
            preferred_element_type=jnp.float32,
        )[:, 0:4]
        s_ref[...] = jnp.zeros_like(s_ref)

    for i in range(_K):
        @pl.when(k == i)
        def _acc(i=i):
            s_ref[...] += jax.lax.dot_general(
                w_ref[:, i * _CHUNK:(i + 1) * _CHUNK], mes_ref[...],
                dimension_numbers=(((1,), (0,)), ((), ())),
                preferred_element_type=jnp.float32,
            )


def kernel(mes_update, yv, yt_pred, cu_seqlens):
    starts = cu_seqlens[:-1].reshape(_B, 1)
    ends = cu_seqlens[1:].reshape(_B, 1)
    ytv = jnp.concatenate([yv.T, yt_pred.T], axis=0)
    s, yv_cas = pl.pallas_call(
        _body,
        grid=(_K,),
        in_specs=[
            pl.BlockSpec((_B, 1), lambda k: (0, 0)),
            pl.BlockSpec((_B, 1), lambda k: (0, 0)),
            pl.BlockSpec((8, _T), lambda k: (0, 0)),
            pl.BlockSpec((_CHUNK, _H), lambda k: (k, 0)),
        ],
        out_specs=(
            pl.BlockSpec((_B, _H), lambda k: (0, 0)),
            pl.BlockSpec((_B, 4), lambda k: (0, 0)),
        ),
        out_shape=(
            jax.ShapeDtypeStruct((_B, _H), jnp.float32),
            jax.ShapeDtypeStruct((_B, 4), jnp.float32),
        ),
        scratch_shapes=[pltpu.VMEM((_B, _T), jnp.float32)],
        compiler_params=pltpu.CompilerParams(
            dimension_semantics=("arbitrary",),
        ),
    )(starts, ends, ytv, mes_update)
    return (s, yv_cas)
